# baseline (speedup 1.0000x reference)
import functools

import jax
import jax.numpy as jnp
from jax.experimental import pallas as pl
from jax.experimental.pallas import tpu as pltpu


def _round_up(x, m):
    return (x + m - 1) // m * m


def _tile_h(s):
    for th in (8, 16, 32):
        if th * _round_up(s + 2, 8) >= 1500 or th == s:
            return min(th, s)
    return min(32, s)


def _rpn_level_kernel(prv_ref, cur_ref, nxt_ref, w1_ref, bsh_ref, wall_ref,
                      ball_ref, o_lg_ref, o_pr_ref, o_rg_ref, slab_ref,
                      *, th, wp, s, nb):
    i = pl.program_id(1)
    m = th * wp
    zc = jnp.zeros((th + 3, 1, 256), dtype=jnp.bfloat16)
    slab_ref[:, 0:1, :] = zc
    slab_ref[:, s + 1:wp, :] = jnp.broadcast_to(zc, (th + 3, wp - s - 1, 256))
    slab_ref[th + 2:th + 3, :, :] = jnp.zeros((1, wp, 256), jnp.bfloat16)
    prv = jnp.where(i > 0, prv_ref[0, 0], 0.0).astype(jnp.bfloat16)
    nxt = jnp.where(i < nb - 1, nxt_ref[0, 0], 0.0).astype(jnp.bfloat16)
    slab_ref[0:1, 1:s + 1, :] = prv[None]
    slab_ref[1:th + 1, 1:s + 1, :] = cur_ref[0].astype(jnp.bfloat16)
    slab_ref[th + 1:th + 2, 1:s + 1, :] = nxt[None]

    slab = slab_ref[...].reshape((th + 3) * wp, 256)
    acc = None
    for dy in range(3):
        for dx in range(3):
            off = dy * wp + dx
            t = jnp.dot(slab[off:off + m, :], w1_ref[dy, dx],
                        preferred_element_type=jnp.float32)
            acc = t if acc is None else acc + t
    shared = jnp.maximum(acc + bsh_ref[:], 0.0)
    out = jnp.dot(shared, wall_ref[:],
                  preferred_element_type=jnp.float32) + ball_ref[:]
    valid = out.reshape(th, wp, 24)[:, 0:s, :].reshape(th * s, 24)
    r3 = th * s * 3
    rep = jnp.broadcast_to(valid[:, None, :], (th * s, 3, 24)).reshape(r3, 24)
    a = jax.lax.broadcasted_iota(jnp.int32, (r3, 1), 0) % 3
    lg = jnp.where(a == 0, rep[:, 0:2],
                   jnp.where(a == 1, rep[:, 2:4], rep[:, 4:6]))
    pr = jnp.where(a == 0, rep[:, 6:8],
                   jnp.where(a == 1, rep[:, 8:10], rep[:, 10:12]))
    rg = jnp.where(a == 0, rep[:, 12:16],
                   jnp.where(a == 1, rep[:, 16:20], rep[:, 20:24]))
    o_lg_ref[0, 0] = lg
    o_pr_ref[0, 0] = jax.nn.sigmoid(pr)
    o_rg_ref[0, 0] = rg


def _rpn_level(x, w1, bsh2, wall, ball2):
    B, s, _, C = x.shape
    wp = _round_up(s + 2, 8)
    th = _tile_h(s)
    nb = s // th
    r3 = th * s * 3

    full = lambda shape: pl.BlockSpec(shape, lambda b, i: (0,) * len(shape))
    ospec = lambda k: pl.BlockSpec((1, 1, r3, k), lambda b, i: (b, i, 0, 0))

    lg, pr, rg = pl.pallas_call(
        functools.partial(_rpn_level_kernel, th=th, wp=wp, s=s, nb=nb),
        grid=(B, nb),
        in_specs=[
            pl.BlockSpec((1, 1, s, C),
                         lambda b, i: (b, jnp.maximum(i * th - 1, 0), 0, 0)),
            pl.BlockSpec((1, th, s, C), lambda b, i: (b, i, 0, 0)),
            pl.BlockSpec((1, 1, s, C),
                         lambda b, i: (b, jnp.minimum(i * th + th, s - 1), 0, 0)),
            full((3, 3, C, 512)),
            full((1, 512)),
            full((512, 24)),
            full((1, 24)),
        ],
        out_specs=[ospec(2), ospec(2), ospec(4)],
        out_shape=[
            jax.ShapeDtypeStruct((B, nb, r3, 2), jnp.float32),
            jax.ShapeDtypeStruct((B, nb, r3, 2), jnp.float32),
            jax.ShapeDtypeStruct((B, nb, r3, 4), jnp.float32),
        ],
        scratch_shapes=[pltpu.VMEM((th + 3, wp, C), jnp.bfloat16)],
    )(x, x, x, w1, bsh2, wall, ball2)
    return (lg.reshape(B, nb * r3, 2), pr.reshape(B, nb * r3, 2),
            rg.reshape(B, nb * r3, 4))


def kernel(feat0, feat1, feat2, feat3, feat4,
           W_shared, b_shared, W_cls, b_cls, W_reg, b_reg):
    wc = W_cls.reshape(512, 6)
    wr = W_reg.reshape(512, 12)
    swap = jnp.array([1, 0, 3, 2, 5, 4], dtype=jnp.int32)
    wdiff = wc - wc[:, swap]
    bdiff = b_cls - b_cls[swap]
    wall = jnp.concatenate([wc, wdiff, wr], axis=1)
    ball2 = jnp.concatenate([b_cls, bdiff, b_reg]).reshape(1, 24)
    bsh2 = b_shared.reshape(1, 512)
    w1 = W_shared.astype(jnp.bfloat16)

    lgs, prs, rgs = [], [], []
    for x in (feat0, feat1, feat2, feat3, feat4):
        lg, pr, rg = _rpn_level(x, w1, bsh2, wall, ball2)
        lgs.append(lg)
        prs.append(pr)
        rgs.append(rg)
    return (jnp.concatenate(lgs, axis=1),
            jnp.concatenate(prs, axis=1),
            jnp.concatenate(rgs, axis=1))


# speedup vs baseline: 1.2513x; 1.2513x over previous
import functools

import jax
import jax.numpy as jnp
import numpy as np
from jax.experimental import pallas as pl
from jax.experimental.pallas import tpu as pltpu


def _round_up(x, m):
    return (x + m - 1) // m * m


def _tile_h(s):
    for th in (8, 16, 32):
        if th * _round_up(s + 2, 8) >= 1500 or th == s:
            return min(th, s)
    return min(32, s)


def _rpn_level_kernel(prv_ref, cur_ref, nxt_ref, w1_ref, bsh_ref, wall_ref,
                      ball_ref, p3_ref, o_lg_ref, o_pr_ref, o_rg_ref, slab_ref,
                      *, th, wp, s, nb):
    i = pl.program_id(1)
    m = th * wp
    zc = jnp.zeros((th + 3, 1, 256), dtype=jnp.bfloat16)
    slab_ref[:, 0:1, :] = zc
    slab_ref[:, s + 1:wp, :] = jnp.broadcast_to(zc, (th + 3, wp - s - 1, 256))
    slab_ref[th + 2:th + 3, :, :] = jnp.zeros((1, wp, 256), jnp.bfloat16)
    prv = jnp.where(i > 0, prv_ref[0, 0], 0.0).astype(jnp.bfloat16)
    nxt = jnp.where(i < nb - 1, nxt_ref[0, 0], 0.0).astype(jnp.bfloat16)
    slab_ref[0:1, 1:s + 1, :] = prv[None]
    slab_ref[1:th + 1, 1:s + 1, :] = cur_ref[0].astype(jnp.bfloat16)
    slab_ref[th + 1:th + 2, 1:s + 1, :] = nxt[None]

    slab = slab_ref[...].reshape((th + 3) * wp, 256)
    acc = None
    for dy in range(3):
        for dx in range(3):
            off = dy * wp + dx
            t = jnp.dot(slab[off:off + m, :], w1_ref[dy, dx],
                        preferred_element_type=jnp.float32)
            acc = t if acc is None else acc + t
    shared = jnp.maximum(acc + bsh_ref[:], 0.0)
    out = jnp.dot(shared, wall_ref[:],
                  preferred_element_type=jnp.float32) + ball_ref[:]
    valid = out.reshape(th, wp, 24)[:, 0:s, :].reshape(th * s, 24)
    r = th * s
    vb = jnp.concatenate(
        [valid[:, 0:2], jax.nn.sigmoid(valid[:, 2:4]), valid[:, 4:10],
         jax.nn.sigmoid(valid[:, 10:12]), valid[:, 12:18],
         jax.nn.sigmoid(valid[:, 18:20]), valid[:, 20:24]],
        axis=1).astype(jnp.bfloat16)
    ch = p3_ref.shape[2]
    chunks = []
    for c in range(r // ch):
        q = vb[c * ch:(c + 1) * ch, :]
        rc = None
        for a in range(3):
            t = jnp.dot(p3_ref[a], q[:, 8 * a:8 * a + 8],
                        preferred_element_type=jnp.float32)
            rc = t if rc is None else rc + t
        chunks.append(rc)
    res = jnp.concatenate(chunks, axis=0) if len(chunks) > 1 else chunks[0]
    o_lg_ref[0, 0] = res[:, 0:2]
    o_pr_ref[0, 0] = res[:, 2:4]
    o_rg_ref[0, 0] = res[:, 4:8]


def _rpn_level(x, w1, bsh2, wall, ball2, p3):
    B, s, _, C = x.shape
    wp = _round_up(s + 2, 8)
    th = _tile_h(s)
    nb = s // th
    r3 = th * s * 3
    ch = min(256, th * s)
    p3l = p3[:, :3 * ch, :ch]

    full = lambda shape: pl.BlockSpec(shape, lambda b, i: (0,) * len(shape))
    ospec = lambda k: pl.BlockSpec((1, 1, r3, k), lambda b, i: (b, i, 0, 0))

    lg, pr, rg = pl.pallas_call(
        functools.partial(_rpn_level_kernel, th=th, wp=wp, s=s, nb=nb),
        grid=(B, nb),
        in_specs=[
            pl.BlockSpec((1, 1, s, C),
                         lambda b, i: (b, jnp.maximum(i * th - 1, 0), 0, 0)),
            pl.BlockSpec((1, th, s, C), lambda b, i: (b, i, 0, 0)),
            pl.BlockSpec((1, 1, s, C),
                         lambda b, i: (b, jnp.minimum(i * th + th, s - 1), 0, 0)),
            full((3, 3, C, 512)),
            full((1, 512)),
            full((512, 24)),
            full((1, 24)),
            full((3, 3 * ch, ch)),
        ],
        out_specs=[ospec(2), ospec(2), ospec(4)],
        out_shape=[
            jax.ShapeDtypeStruct((B, nb, r3, 2), jnp.float32),
            jax.ShapeDtypeStruct((B, nb, r3, 2), jnp.float32),
            jax.ShapeDtypeStruct((B, nb, r3, 4), jnp.float32),
        ],
        scratch_shapes=[pltpu.VMEM((th + 3, wp, C), jnp.bfloat16)],
    )(x, x, x, w1, bsh2, wall, ball2, p3l)
    return (lg.reshape(B, nb * r3, 2), pr.reshape(B, nb * r3, 2),
            rg.reshape(B, nb * r3, 4))


def kernel(feat0, feat1, feat2, feat3, feat4,
           W_shared, b_shared, W_cls, b_cls, W_reg, b_reg):
    wc = W_cls.reshape(512, 6)
    wr = W_reg.reshape(512, 12)
    swap = jnp.array([1, 0, 3, 2, 5, 4], dtype=jnp.int32)
    wdiff = wc - wc[:, swap]
    bdiff = b_cls - b_cls[swap]
    wall = jnp.concatenate(
        [jnp.concatenate([wc[:, 2 * a:2 * a + 2], wdiff[:, 2 * a:2 * a + 2],
                          wr[:, 4 * a:4 * a + 4]], axis=1) for a in range(3)],
        axis=1)
    ball2 = jnp.concatenate(
        [jnp.concatenate([b_cls[2 * a:2 * a + 2], bdiff[2 * a:2 * a + 2],
                          b_reg[4 * a:4 * a + 4]]) for a in range(3)]
    ).reshape(1, 24)
    bsh2 = b_shared.reshape(1, 512)
    w1 = W_shared.astype(jnp.bfloat16)
    chmax = 256
    p3np = np.zeros((3, 3 * chmax, chmax), dtype=np.float32)
    k = np.arange(chmax)
    for a in range(3):
        p3np[a, 3 * k + a, k] = 1.0
    p3 = jnp.asarray(p3np, dtype=jnp.bfloat16)

    lgs, prs, rgs = [], [], []
    for x in (feat0, feat1, feat2, feat3, feat4):
        lg, pr, rg = _rpn_level(x, w1, bsh2, wall, ball2, p3)
        lgs.append(lg)
        prs.append(pr)
        rgs.append(rg)
    return (jnp.concatenate(lgs, axis=1),
            jnp.concatenate(prs, axis=1),
            jnp.concatenate(rgs, axis=1))


# speedup vs baseline: 1.8717x; 1.4958x over previous
import functools

import jax
import jax.numpy as jnp
import numpy as np
from jax.experimental import pallas as pl
from jax.experimental.pallas import tpu as pltpu


def _round_up(x, m):
    return (x + m - 1) // m * m


def _tile_h(s):
    for th in (8, 16, 32):
        if th * _round_up(s + 2, 8) >= 1500 or th == s:
            return min(th, s)
    return min(32, s)


def _rpn_level_kernel(prv_ref, cur_ref, nxt_ref, w1_ref, bsh_ref, wall_ref,
                      ball_ref, p3_ref, o_ref, slab_ref, *, th, wp, s, nb):
    i = pl.program_id(1)
    m = th * wp
    zc = jnp.zeros((th + 3, 1, 256), dtype=jnp.bfloat16)
    slab_ref[:, 0:1, :] = zc
    slab_ref[:, s + 1:wp, :] = jnp.broadcast_to(zc, (th + 3, wp - s - 1, 256))
    slab_ref[th + 2:th + 3, :, :] = jnp.zeros((1, wp, 256), jnp.bfloat16)
    prv = jnp.where(i > 0, prv_ref[0, 0], 0.0).astype(jnp.bfloat16)
    nxt = jnp.where(i < nb - 1, nxt_ref[0, 0], 0.0).astype(jnp.bfloat16)
    slab_ref[0:1, 1:s + 1, :] = prv[None]
    slab_ref[1:th + 1, 1:s + 1, :] = cur_ref[0].astype(jnp.bfloat16)
    slab_ref[th + 1:th + 2, 1:s + 1, :] = nxt[None]

    slab = slab_ref[...].reshape((th + 3) * wp, 256)
    acc = None
    for dy in range(3):
        for dx in range(3):
            off = dy * wp + dx
            t = jnp.dot(slab[off:off + m, :], w1_ref[dy, dx],
                        preferred_element_type=jnp.float32)
            acc = t if acc is None else acc + t
    shared = jnp.maximum(acc + bsh_ref[:], 0.0)
    out = jnp.dot(shared, wall_ref[:],
                  preferred_element_type=jnp.float32) + ball_ref[:]
    valid = out.reshape(th, wp, 24)[:, 0:s, :].reshape(th * s, 24)
    r = th * s
    vb = jnp.concatenate(
        [valid[:, 0:2], jax.nn.sigmoid(valid[:, 2:4]), valid[:, 4:10],
         jax.nn.sigmoid(valid[:, 10:12]), valid[:, 12:18],
         jax.nn.sigmoid(valid[:, 18:20]), valid[:, 20:24]],
        axis=1).astype(jnp.bfloat16)
    ch = p3_ref.shape[2]
    chunks = []
    for c in range(r // ch):
        q = vb[c * ch:(c + 1) * ch, :]
        rc = None
        for a in range(3):
            t = jnp.dot(p3_ref[a], q[:, 8 * a:8 * a + 8],
                        preferred_element_type=jnp.float32)
            rc = t if rc is None else rc + t
        chunks.append(rc)
    res = jnp.concatenate(chunks, axis=0) if len(chunks) > 1 else chunks[0]
    o_ref[0, 0] = res


def _rpn_level(x, w1, bsh2, wall, ball2, p3):
    B, s, _, C = x.shape
    wp = _round_up(s + 2, 8)
    th = _tile_h(s)
    nb = s // th
    r3 = th * s * 3
    ch = min(128, th * s)
    p3l = p3[:, :3 * ch, :ch]

    full = lambda shape: pl.BlockSpec(shape, lambda b, i: (0,) * len(shape))
    ospec = lambda k: pl.BlockSpec((1, 1, r3, k), lambda b, i: (b, i, 0, 0))

    res = pl.pallas_call(
        functools.partial(_rpn_level_kernel, th=th, wp=wp, s=s, nb=nb),
        grid=(B, nb),
        in_specs=[
            pl.BlockSpec((1, 1, s, C),
                         lambda b, i: (b, jnp.maximum(i * th - 1, 0), 0, 0)),
            pl.BlockSpec((1, th, s, C), lambda b, i: (b, i, 0, 0)),
            pl.BlockSpec((1, 1, s, C),
                         lambda b, i: (b, jnp.minimum(i * th + th, s - 1), 0, 0)),
            full((3, 3, C, 512)),
            full((1, 512)),
            full((512, 24)),
            full((1, 24)),
            full((3, 3 * ch, ch)),
        ],
        out_specs=ospec(8),
        out_shape=jax.ShapeDtypeStruct((B, nb, r3, 8), jnp.float32),
        scratch_shapes=[pltpu.VMEM((th + 3, wp, C), jnp.bfloat16)],
    )(x, x, x, w1, bsh2, wall, ball2, p3l)
    return res.reshape(B, nb * r3, 8)


def kernel(feat0, feat1, feat2, feat3, feat4,
           W_shared, b_shared, W_cls, b_cls, W_reg, b_reg):
    wc = W_cls.reshape(512, 6)
    wr = W_reg.reshape(512, 12)
    swap = jnp.array([1, 0, 3, 2, 5, 4], dtype=jnp.int32)
    wdiff = wc - wc[:, swap]
    bdiff = b_cls - b_cls[swap]
    wall = jnp.concatenate(
        [jnp.concatenate([wc[:, 2 * a:2 * a + 2], wdiff[:, 2 * a:2 * a + 2],
                          wr[:, 4 * a:4 * a + 4]], axis=1) for a in range(3)],
        axis=1)
    ball2 = jnp.concatenate(
        [jnp.concatenate([b_cls[2 * a:2 * a + 2], bdiff[2 * a:2 * a + 2],
                          b_reg[4 * a:4 * a + 4]]) for a in range(3)]
    ).reshape(1, 24)
    bsh2 = b_shared.reshape(1, 512)
    w1 = W_shared.astype(jnp.bfloat16)
    chmax = 128
    p3np = np.zeros((3, 3 * chmax, chmax), dtype=np.float32)
    k = np.arange(chmax)
    for a in range(3):
        p3np[a, 3 * k + a, k] = 1.0
    p3 = jnp.asarray(p3np, dtype=jnp.bfloat16)

    outs = [_rpn_level(x, w1, bsh2, wall, ball2, p3)
            for x in (feat0, feat1, feat2, feat3, feat4)]
    big = jnp.concatenate(outs, axis=1)
    return (big[..., 0:2], big[..., 2:4], big[..., 4:8])
